# Initial kernel scaffold; baseline (speedup 1.0000x reference)
import jax
import jax.numpy as jnp
from jax.experimental import pallas as pl


def kernel(x, co_edge_weight, go_edge_weight, bg_edge_weight, params, pert_idx, co_edge_index, go_edge_index, bg_edge_index):
    raise NotImplementedError("write your pallas kernel here")


# speedup vs baseline: 4.8191x; 4.8191x over previous
import jax
import jax.numpy as jnp
from jax import lax
from jax.experimental import pallas as pl
from jax.experimental.pallas import tpu as pltpu

BB = 16
GG = 5000
PP = 5000
HH = 64
NP = 5120
EPS = 1e-5


def _prep_body(tabs_ref, degraw_ref, r01_ref, yS_ref, dinv_ref):
    rs = []
    for t in range(4):
        tab = tabs_ref[t]
        n = jnp.sqrt(jnp.sum(tab * tab, axis=1, keepdims=True))
        r = tab * jnp.minimum(1.0, 1.0 / (n + 1e-7))
        rs.append(r)
        if t >= 1:
            dv = lax.rsqrt(degraw_ref[t - 1] + 1.0)
            dinv_ref[t - 1] = dv
            yS_ref[t - 1] = jnp.concatenate(
                [r * dv[:, None], jnp.zeros((NP, HH), jnp.float32)], axis=1)
    r01_ref[...] = jnp.concatenate(rs[:2], axis=1)


def _prep(tabs, degraw):
    return pl.pallas_call(
        _prep_body,
        out_shape=[
            jax.ShapeDtypeStruct((NP, 2 * HH), jnp.float32),
            jax.ShapeDtypeStruct((3, NP, 2 * HH), jnp.float32),
            jax.ShapeDtypeStruct((3, NP), jnp.float32),
        ],
    )(tabs, degraw)


def _bn_affine(s, q, n, g, be):
    m = s / n
    v = q / n - m * m
    a = g * lax.rsqrt(v + EPS)
    return a, be - m * a


def _main_body(r01, yS3, dinv3,
               sgWt, sgb,
               bn_emb, bn_pb,
               v2_W1t, v2_vec1, v2_W2t, v2_vec2,
               comb_W1t, comb_vec1, comb_W2t, comb_vec2,
               fuse_W1t, fuse_vec1, fuse_W2t, fuse_vec2,
               rec_W1t, rec_vec1, rec_W2t, rec_vec2,
               cgs_vec1, cgs_W2t, cgs_vec2,
               wcg, b1g, w2a, w2bT, b2g,
               xg, pert_idx,
               out_ref, pglob_s, e_s, d1_s):
    msk = (lax.broadcasted_iota(jnp.int32, (NP, 1), 0) < GG).astype(jnp.float32)
    n1 = float(GG)
    nb = float(BB * GG)

    (sgWt, sgb, bn_emb, bn_pb,
     v2_W1t, v2_vec1, v2_W2t, v2_vec2,
     comb_W1t, comb_vec1, comb_W2t, comb_vec2,
     fuse_W1t, fuse_vec1, fuse_W2t, fuse_vec2,
     rec_W1t, rec_vec1, rec_W2t, rec_vec2,
     cgs_vec1, cgs_W2t, cgs_vec2, w2bT) = [
        ref[...] for ref in (
            sgWt, sgb, bn_emb, bn_pb,
            v2_W1t, v2_vec1, v2_W2t, v2_vec2,
            comb_W1t, comb_vec1, comb_W2t, comb_vec2,
            fuse_W1t, fuse_vec1, fuse_W2t, fuse_vec2,
            rec_W1t, rec_vec1, rec_W2t, rec_vec2,
            cgs_vec1, cgs_W2t, cgs_vec2, w2bT)]

    rg = r01[:, :HH]
    s = jnp.sum(rg, 0)
    q = jnp.sum(rg * rg, 0)
    a, c = _bn_affine(s, q, n1, bn_emb[0], bn_emb[1])
    base0 = jax.nn.relu(rg * a + c) * msk

    aggs = []
    for g in range(3):
        dv = dinv3[g][:, None]
        agg = dv * (yS3[g, :, :HH] + yS3[g, :, HH:])
        aggs.append((jnp.dot(agg, sgWt[g], preferred_element_type=jnp.float32)
                     + sgb[g][None, :]) * msk)
    posA = aggs[0]
    posR = (jnp.dot(r01[:, HH:], sgWt[0], preferred_element_type=jnp.float32)
            + sgb[0][None, :]) * msk
    baseA = base0 + 0.2 * posA
    baseR = base0 + 0.2 * posR

    def mlp2_variants(xs, wts, ntot, W1t, vec1, W2t, vec2):
        zs = [(jnp.dot(x, W1t, preferred_element_type=jnp.float32)
               + vec1[0][None, :]) * msk for x in xs]
        s1 = sum(w * jnp.sum(z, 0) for w, z in zip(wts, zs))
        q1 = sum(w * jnp.sum(z * z, 0) for w, z in zip(wts, zs))
        a1, c1 = _bn_affine(s1, q1, ntot, vec1[1], vec1[2])
        hs = [jax.nn.relu(z * a1[None, :] + c1[None, :]) * msk for z in zs]
        z2s = [(jnp.dot(h, W2t, preferred_element_type=jnp.float32)
                + vec2[0][None, :]) * msk for h in hs]
        s2 = sum(w * jnp.sum(z, 0) for w, z in zip(wts, z2s))
        q2 = sum(w * jnp.sum(z * z, 0) for w, z in zip(wts, z2s))
        a2, c2 = _bn_affine(s2, q2, ntot, vec2[1], vec2[2])
        return [(z * a2[None, :] + c2[None, :]) * msk for z in z2s]

    vA, vR = mlp2_variants([baseA, baseR], [1.0, 15.0], nb,
                           v2_W1t, v2_vec1, v2_W2t, v2_vec2)

    pglob = mlp2_variants([aggs[1] + aggs[2]], [1.0], n1,
                          comb_W1t, comb_vec1, comb_W2t, comb_vec2)[0]
    pglob_s[...] = pglob
    rows = []
    for b in range(BB):
        idx = pert_idx[b, 0]
        rows.append(pglob_s[pl.ds(idx, 1), :])
    track = jnp.concatenate(rows, axis=0)

    z = jnp.dot(track, fuse_W1t, preferred_element_type=jnp.float32) + fuse_vec1[0][None, :]
    a1, c1 = _bn_affine(jnp.sum(z, 0), jnp.sum(z * z, 0), float(BB),
                        fuse_vec1[1], fuse_vec1[2])
    h = jax.nn.relu(z * a1[None, :] + c1[None, :])
    z2 = jnp.dot(h, fuse_W2t, preferred_element_type=jnp.float32) + fuse_vec2[0][None, :]
    a2, c2 = _bn_affine(jnp.sum(z2, 0), jnp.sum(z2 * z2, 0), float(BB),
                        fuse_vec2[1], fuse_vec2[2])
    e = z2 * a2[None, :] + c2[None, :]
    e_s[...] = e

    sA = jnp.sum(vA, 0); sR = jnp.sum(vR, 0)
    qA = jnp.sum(vA * vA, 0); qR = jnp.sum(vR * vR, 0)
    e0 = e[0]
    er = e[1:, :]
    se_r = jnp.sum(er, 0)
    s = sA + 15.0 * sR + GG * (e0 + se_r)
    q = (qA + 2.0 * e0 * sA + GG * e0 * e0
         + 15.0 * qR + 2.0 * se_r * sR + GG * jnp.sum(er * er, 0))
    a_bn, c_bn = _bn_affine(s, q, nb, bn_pb[0], bn_pb[1])

    def xb(b):
        eb = e_s[pl.ds(b, 1), :]
        vb = jnp.where(b == 0, vA, vR)
        return jax.nn.relu((vb + eb) * a_bn[None, :] + c_bn[None, :]) * msk

    def p1(b, carry):
        xsum, Gt = carry
        x = xb(b)
        return (xsum + jnp.sum(x, 0),
                Gt + lax.dot_general(x, x, (((0,), (0,)), ((), ())),
                                     preferred_element_type=jnp.float32))
    xsum, Gtot = lax.fori_loop(
        0, BB, p1, (jnp.zeros((HH,), jnp.float32),
                    jnp.zeros((HH, HH), jnp.float32)))
    b1 = rec_vec1[0]
    sw = jnp.dot(xsum[None, :], rec_W1t, preferred_element_type=jnp.float32)[0]
    s1 = sw + nb * b1
    GW = jnp.dot(Gtot, rec_W1t, preferred_element_type=jnp.float32)
    q1 = jnp.sum(GW * rec_W1t, 0) + 2.0 * b1 * sw + nb * b1 * b1
    a1, c1 = _bn_affine(s1, q1, nb, rec_vec1[1], rec_vec1[2])
    c1p = (b1 * a1 + c1)[None, :]

    def tb(b):
        x = xb(b)
        z1 = jnp.dot(x, rec_W1t, preferred_element_type=jnp.float32)
        h = jax.nn.relu(z1 * a1[None, :] + c1p) * msk
        return jnp.dot(h, rec_W2t, preferred_element_type=jnp.float32)

    def p2(b, carry):
        st, qt = carry
        t = tb(b)
        return st + jnp.sum(t, 0), qt + jnp.sum(t * t, 0)
    st, qt = lax.fori_loop(
        0, BB, p2, (jnp.zeros((HH,), jnp.float32),
                    jnp.zeros((HH,), jnp.float32)))
    b2 = rec_vec2[0]
    s2 = st + nb * b2
    q2 = qt + 2.0 * b2 * st + nb * b2 * b2
    a2, c2 = _bn_affine(s2, q2, nb, rec_vec2[1], rec_vec2[2])
    c2p = b2 * a2 + c2

    w1g = wcg[:, :HH]
    cgs_W1t = wcg[:, HH:]
    w1g2 = w1g * a2[None, :]
    kvec = jnp.sum(w1g * c2p[None, :], 1) + b1g[...]

    def p3(b, _):
        t = tb(b)
        d1_s[pl.ds(b, 1), :] = (jnp.sum(t * w1g2, 1) + kvec)[None, :]
        return 0
    lax.fori_loop(0, BB, p3, 0)
    d1 = d1_s[...]

    z = (lax.dot_general(d1, cgs_W1t, (((1,), (0,)), ((), ())),
                         preferred_element_type=jnp.float32)
         + cgs_vec1[0][None, :])
    a1, c1 = _bn_affine(jnp.sum(z, 0), jnp.sum(z * z, 0), float(BB),
                        cgs_vec1[1], cgs_vec1[2])
    h = jax.nn.relu(z * a1[None, :] + c1[None, :])
    z2 = jnp.dot(h, cgs_W2t, preferred_element_type=jnp.float32) + cgs_vec2[0][None, :]
    a2, c2 = _bn_affine(jnp.sum(z2, 0), jnp.sum(z2 * z2, 0), float(BB),
                        cgs_vec2[1], cgs_vec2[2])
    cge = z2 * a2[None, :] + c2[None, :]

    cro = (d1 * w2a[...][None, :]
           + jnp.dot(cge, w2bT, preferred_element_type=jnp.float32)
           + b2g[...][None, :])
    out_ref[...] = cro[:, :GG] + xg[...]


def _main(ops):
    in_specs = [pl.BlockSpec(memory_space=pltpu.VMEM) for _ in ops]
    in_specs[-1] = pl.BlockSpec(memory_space=pltpu.SMEM)
    return pl.pallas_call(
        _main_body,
        out_shape=jax.ShapeDtypeStruct((BB, GG), jnp.float32),
        in_specs=in_specs,
        scratch_shapes=[
            pltpu.VMEM((NP, HH), jnp.float32),
            pltpu.VMEM((BB, HH), jnp.float32),
            pltpu.VMEM((BB, NP), jnp.float32),
        ],
    )(*ops)


def kernel(x, co_edge_weight, go_edge_weight, bg_edge_weight, params,
           pert_idx, co_edge_index, go_edge_index, bg_edge_index):
    p = params
    f32 = jnp.float32

    def padn(a):
        return jnp.pad(a, ((0, NP - a.shape[0]),) + ((0, 0),) * (a.ndim - 1))

    tabs = jnp.stack([padn(p['gene_emb']), padn(p['emb_pos']),
                      padn(p['pert_go_tab']), padn(p['pert_bg_tab'])])

    eis = [co_edge_index.astype(jnp.int32), go_edge_index.astype(jnp.int32),
           bg_edge_index.astype(jnp.int32)]
    ews = [co_edge_weight, go_edge_weight, bg_edge_weight]

    degraw = jnp.stack([
        jnp.zeros((NP,), f32).at[eis[g][1]].add(ews[g]) for g in range(3)])

    r01, yS3, dinv3 = _prep(tabs, degraw)

    S3 = jnp.stack([
        jnp.zeros((NP, HH), f32).at[eis[g][1]].add(
            yS3[g, :, :HH][eis[g][0]] * ews[g][:, None]) for g in range(3)])
    yS3 = yS3.at[:, :, HH:].set(S3)

    def mlp_ops(mp):
        return [mp['W1'].T, jnp.stack([mp['b1'], mp['g1'], mp['be1']]),
                mp['W2'].T, jnp.stack([mp['b2'], mp['g2'], mp['be2']])]

    cgs = p['cgs']
    cgs_W1t_pad = jnp.pad(cgs['W1'], ((0, 0), (0, NP - GG))).T
    wcg = jnp.concatenate([padn(p['indv_w1'][:, :, 0]), cgs_W1t_pad], axis=1)

    ops = [r01, yS3, dinv3,
           jnp.stack([p['sg_pos_W'].T, p['sg_go_W'].T, p['sg_bg_W'].T]),
           jnp.stack([p['sg_pos_b'], p['sg_go_b'], p['sg_bg_b']]),
           jnp.stack([p['bn_emb_g'], p['bn_emb_b']]),
           jnp.stack([p['bn_pb_g'], p['bn_pb_b']]),
           *mlp_ops(p['v2']), *mlp_ops(p['comb']), *mlp_ops(p['fuse']),
           *mlp_ops(p['rec']),
           jnp.stack([cgs['b1'], cgs['g1'], cgs['be1']]), cgs['W2'].T,
           jnp.stack([cgs['b2'], cgs['g2'], cgs['be2']]),
           wcg, padn(p['indv_b1'][:, 0]),
           padn(p['indv_w2'][0, :, 0]), padn(p['indv_w2'][0, :, 1:]).T,
           padn(p['indv_b2'][0]),
           x.reshape(BB, GG), pert_idx.astype(jnp.int32)]

    return _main(ops)
